# baseline (speedup 1.0000x reference)
import functools

import jax
import jax.numpy as jnp
from jax import lax
from jax.experimental import pallas as pl
from jax.experimental.pallas import tpu as pltpu
from jax.experimental.pallas import tpu_sc as plsc

_LANES = 16
_NWORKERS = 32
_CHUNK = 16
_QUAD = 2
_LN_EPS = 1e-12

_GATHER_DNUMS = lax.GatherDimensionNumbers(
    offset_dims=(), collapsed_slice_dims=(0,), start_index_map=(0,))


def _lane_shuffle(x, idx):
    return lax.gather(x, idx[:, None], _GATHER_DNUMS, slice_sizes=(1,),
                      mode=lax.GatherScatterMode.PROMISE_IN_BOUNDS)


def _allreduce16(x):
    iota = lax.iota(jnp.int32, _LANES)
    for sh in (8, 4, 2, 1):
        x = x + _lane_shuffle(x, iota ^ sh)
    return x


def _rsqrt16(x):
    i = plsc.bitcast(x, jnp.int32)
    y = plsc.bitcast(jnp.int32(0x5F3759DF) - (i >> 1), jnp.float32)
    half_x = x * jnp.float32(0.5)
    y = y * (jnp.float32(1.5) - half_x * y * y)
    y = y * (jnp.float32(1.5) - half_x * y * y)
    y = y * (jnp.float32(1.5) - half_x * y * y)
    return y


@functools.lru_cache(maxsize=None)
def _build(n_tok, seq_len, hidden):
    spw = n_tok // _NWORKERS
    n_pairs = spw // (2 * _CHUNK)
    nv = hidden // _LANES
    mesh = plsc.VectorSubcoreMesh(core_axis_name="c", subcore_axis_name="s")
    buf_t = pltpu.VMEM((_CHUNK, hidden), jnp.float32)
    vec_t = pltpu.VMEM((hidden,), jnp.float32)

    @functools.partial(
        pl.kernel,
        out_type=jax.ShapeDtypeStruct((n_tok, hidden), jnp.float32),
        mesh=mesh,
        compiler_params=pltpu.CompilerParams(needs_layout_passes=False),
        scratch_types=[
            pltpu.VMEM((spw,), jnp.int32),
            pltpu.VMEM((spw,), jnp.int32),
            pltpu.VMEM((spw,), jnp.int32),
            buf_t, buf_t,
            buf_t, buf_t,
            pltpu.VMEM((2, hidden), jnp.float32),
            pltpu.VMEM((3, hidden), jnp.float32),
            vec_t, vec_t, vec_t, vec_t,
            vec_t, vec_t,
            pltpu.SemaphoreType.DMA,
            pltpu.SemaphoreType.DMA,
        ],
    )
    def tie_kernel(ids_hbm, tt_hbm, task_hbm, wemb, pemb, temb, kemb,
                   gamma_hbm, beta_hbm, out_hbm,
                   ids_v, tt_v, task_v,
                   wa, pa, wb, pb,
                   ttab, ktab, base_v, d01_v, kd1_v, kd2_v,
                   gamma_v, beta_v, sem_g, sem_o):
        wid = lax.axis_index("s") * mesh.num_cores + lax.axis_index("c")
        base = wid * spw
        s_base = lax.rem(base, seq_len)

        pltpu.sync_copy(ids_hbm.at[pl.ds(base, spw)], ids_v)
        pltpu.sync_copy(tt_hbm.at[pl.ds(base, spw)], tt_v)
        pltpu.sync_copy(task_hbm.at[pl.ds(base, spw)], task_v)
        pltpu.sync_copy(gamma_hbm, gamma_v)
        pltpu.sync_copy(beta_hbm, beta_v)
        pltpu.sync_copy(temb, ttab)
        pltpu.sync_copy(kemb, ktab)

        two = jnp.float32(2.0)
        for j in range(nv):
            sl = pl.ds(j * _LANES, _LANES)
            t0 = ttab[0, sl]
            t1 = ttab[1, sl]
            k0 = ktab[0, sl]
            k1 = ktab[1, sl]
            k2 = ktab[2, sl]
            base_v[sl] = t0 + k0
            d01_v[sl] = t1 - t0
            kd1_v[sl] = k1 - k0
            kd2_v[sl] = k2 - two * k1 + k0

        def issue(c, w, p):
            off = pl.multiple_of(c * _CHUNK, _CHUNK)
            pltpu.async_copy(wemb.at[ids_v[pl.ds(off, _CHUNK)]], w, sem_g)
            pltpu.async_copy(pemb.at[pl.ds(s_base + off, _CHUNK)], p, sem_g)

        def wait_gathers(w):
            for _ in range(2):
                pltpu.make_async_copy(pemb.at[pl.ds(0, _CHUNK)], w,
                                      sem_g).wait()

        def drain_out(w):
            pltpu.make_async_copy(pemb.at[pl.ds(0, _CHUNK)], w, sem_o).wait()

        zz = jnp.zeros((_LANES,), jnp.float32)
        inv_h = jnp.float32(1.0 / hidden)
        half = jnp.float32(0.5)
        one = jnp.float32(1.0)

        def compute(c, w, p):
            off = pl.multiple_of(c * _CHUNK, _CHUNK)
            tt16 = tt_v[pl.ds(off, _CHUNK)].astype(jnp.float32)
            kk16 = task_v[pl.ds(off, _CHUNK)].astype(jnp.float32)

            def quad_body(q, carry):
                t0 = q * _QUAD
                bcast = []
                for dt in range(_QUAD):
                    splat = jnp.full((_LANES,), t0 + dt, jnp.int32)
                    ttf = _lane_shuffle(tt16, splat)
                    kf = _lane_shuffle(kk16, splat)
                    c2 = kf * (kf - one) * half
                    bcast.append((ttf, kf, c2))
                accs = [[zz, zz] for _ in range(_QUAD)]
                for j in range(nv):
                    sl = pl.ds(j * _LANES, _LANES)
                    vb = base_v[sl]
                    vd = d01_v[sl]
                    v1 = kd1_v[sl]
                    v2 = kd2_v[sl]
                    for dt in range(_QUAD):
                        tk = t0 + dt
                        ttf, kf, c2 = bcast[dt]
                        v = (w[tk, sl] + p[tk, sl] + vb
                             + ttf * vd + kf * v1 + c2 * v2)
                        w[tk, sl] = v
                        accs[dt][0] = accs[dt][0] + v
                        accs[dt][1] = accs[dt][1] + v * v
                stats = []
                for dt in range(_QUAD):
                    mean_v = _allreduce16(accs[dt][0]) * inv_h
                    var_v = (_allreduce16(accs[dt][1]) * inv_h
                             - mean_v * mean_v)
                    rstd_v = _rsqrt16(var_v + jnp.float32(_LN_EPS))
                    stats.append((mean_v, rstd_v))
                for j in range(nv):
                    sl = pl.ds(j * _LANES, _LANES)
                    g = gamma_v[sl]
                    b = beta_v[sl]
                    for dt in range(_QUAD):
                        tk = t0 + dt
                        mean_v, rstd_v = stats[dt]
                        a = g * rstd_v
                        w[tk, sl] = (w[tk, sl] - mean_v) * a + b
                return carry

            lax.fori_loop(0, _CHUNK // _QUAD, quad_body, 0)
            pltpu.async_copy(w, out_hbm.at[pl.ds(base + off, _CHUNK)], sem_o)

        issue(0, wa, pa)

        def pair_body(cp, carry):
            c0 = cp * 2
            wait_gathers(wa)

            @pl.when(cp > 0)
            def _():
                drain_out(wb)

            issue(c0 + 1, wb, pb)
            compute(c0, wa, pa)
            wait_gathers(wb)

            @pl.when(cp + 1 < n_pairs)
            def _():
                drain_out(wa)
                issue(c0 + 2, wa, pa)

            compute(c0 + 1, wb, pb)
            return carry

        lax.fori_loop(0, n_pairs, pair_body, 0)
        drain_out(wa)
        drain_out(wb)

    return tie_kernel


def kernel(input_ids, token_type_ids, task_type_ids, word_emb, pos_emb,
           type_emb, task_emb, ln_gamma, ln_beta):
    b, s = input_ids.shape
    hidden = word_emb.shape[1]
    n_tok = b * s
    fn = _build(n_tok, s, hidden)
    out = fn(input_ids.reshape(-1).astype(jnp.int32),
             token_type_ids.reshape(-1).astype(jnp.int32),
             task_type_ids.reshape(-1).astype(jnp.int32),
             word_emb, pos_emb, type_emb, task_emb, ln_gamma, ln_beta)
    return out.reshape(b, s, hidden)


# speedup vs baseline: 2.4135x; 2.4135x over previous
import functools

import jax
import jax.numpy as jnp
from jax import lax
from jax.experimental import pallas as pl
from jax.experimental.pallas import tpu as pltpu
from jax.experimental.pallas import tpu_sc as plsc

_LANES = 16
_NWORKERS = 32
_CHUNK = 16
_QUAD = 2
_LN_EPS = 1e-12

_GATHER_DNUMS = lax.GatherDimensionNumbers(
    offset_dims=(), collapsed_slice_dims=(0,), start_index_map=(0,))


def _lane_shuffle(x, idx):
    return lax.gather(x, idx[:, None], _GATHER_DNUMS, slice_sizes=(1,),
                      mode=lax.GatherScatterMode.PROMISE_IN_BOUNDS)


def _allreduce16(x):
    iota = lax.iota(jnp.int32, _LANES)
    for sh in (8, 4, 2, 1):
        x = x + _lane_shuffle(x, iota ^ sh)
    return x


def _rsqrt16(x):
    i = plsc.bitcast(x, jnp.int32)
    y = plsc.bitcast(jnp.int32(0x5F3759DF) - (i >> 1), jnp.float32)
    half_x = x * jnp.float32(0.5)
    y = y * (jnp.float32(1.5) - half_x * y * y)
    y = y * (jnp.float32(1.5) - half_x * y * y)
    y = y * (jnp.float32(1.5) - half_x * y * y)
    return y


@functools.lru_cache(maxsize=None)
def _build(n_tok, seq_len, hidden):
    spw = n_tok // _NWORKERS
    n_pairs = spw // (2 * _CHUNK)
    nv = hidden // _LANES
    mesh = plsc.VectorSubcoreMesh(core_axis_name="c", subcore_axis_name="s")
    buf_t = pltpu.VMEM((_CHUNK, hidden), jnp.float32)
    vec_t = pltpu.VMEM((hidden,), jnp.float32)

    @functools.partial(
        pl.kernel,
        out_type=jax.ShapeDtypeStruct((n_tok, hidden), jnp.float32),
        mesh=mesh,
        compiler_params=pltpu.CompilerParams(needs_layout_passes=False),
        scratch_types=[
            pltpu.VMEM((spw,), jnp.int32),
            pltpu.VMEM((spw,), jnp.int32),
            pltpu.VMEM((spw,), jnp.int32),
            pltpu.VMEM((spw + _LANES,), jnp.int32),
            buf_t, buf_t,
            buf_t, buf_t,
            pltpu.VMEM((2, hidden), jnp.float32),
            pltpu.VMEM((3, hidden), jnp.float32),
            pltpu.VMEM((6, hidden), jnp.float32),
            vec_t, vec_t,
            pltpu.SemaphoreType.DMA,
            pltpu.SemaphoreType.DMA,
        ],
    )
    def tie_kernel(ids_hbm, tt_hbm, task_hbm, wemb, pemb, temb, kemb,
                   gamma_hbm, beta_hbm, out_hbm,
                   ids_v, tt_v, task_v, cid_v,
                   wa, pa, wb, pb,
                   ttab, ktab, comb, gamma_v, beta_v, sem_g, sem_o):
        wid = lax.axis_index("s") * mesh.num_cores + lax.axis_index("c")
        base = wid * spw
        s_base = lax.rem(base, seq_len)

        pltpu.sync_copy(ids_hbm.at[pl.ds(base, spw)], ids_v)
        pltpu.sync_copy(tt_hbm.at[pl.ds(base, spw)], tt_v)
        pltpu.sync_copy(task_hbm.at[pl.ds(base, spw)], task_v)
        pltpu.sync_copy(gamma_hbm, gamma_v)
        pltpu.sync_copy(beta_hbm, beta_v)
        pltpu.sync_copy(temb, ttab)
        pltpu.sync_copy(kemb, ktab)

        three = jnp.full((_LANES,), 3, jnp.int32)
        zzi = jnp.zeros((_LANES,), jnp.int32)

        def cid_body(i, carry):
            sl = pl.ds(i * _LANES, _LANES)
            cid_v[sl] = tt_v[sl] * three + task_v[sl]
            return carry

        lax.fori_loop(0, spw // _LANES, cid_body, 0)
        cid_v[pl.ds(spw, _LANES)] = zzi

        def comb_body(j, carry):
            sl = pl.ds(j * _LANES, _LANES)
            for r in range(2):
                t_row = ttab[r, sl]
                for kk in range(3):
                    comb[r * 3 + kk, sl] = t_row + ktab[kk, sl]
            return carry

        lax.fori_loop(0, nv, comb_body, 0)

        def issue(c, w, p):
            off = pl.multiple_of(c * _CHUNK, _CHUNK)
            pltpu.async_copy(wemb.at[ids_v[pl.ds(off, _CHUNK)]], w, sem_g)
            pltpu.async_copy(pemb.at[pl.ds(s_base + off, _CHUNK)], p, sem_g)

        def wait_gathers(w):
            for _ in range(2):
                pltpu.make_async_copy(pemb.at[pl.ds(0, _CHUNK)], w,
                                      sem_g).wait()

        def drain_out(w):
            pltpu.make_async_copy(pemb.at[pl.ds(0, _CHUNK)], w, sem_o).wait()

        zz = jnp.zeros((_LANES,), jnp.float32)
        inv_h = jnp.float32(1.0 / hidden)

        def compute(c, w, p):
            off = pl.multiple_of(c * _CHUNK, _CHUNK)

            def quad_body(q, carry):
                t0 = q * _QUAD
                stats = []
                for dt in range(_QUAD):
                    tk = t0 + dt
                    cid = cid_v[pl.ds(off + tk, _LANES)][0]
                    s = zz
                    ss = zz
                    for j in range(nv):
                        sl = pl.ds(j * _LANES, _LANES)
                        v = w[tk, sl] + p[tk, sl] + comb[cid, sl]
                        w[tk, sl] = v
                        s = s + v
                        ss = ss + v * v
                    mean_v = _allreduce16(s) * inv_h
                    var_v = _allreduce16(ss) * inv_h - mean_v * mean_v
                    rstd_v = _rsqrt16(var_v + jnp.float32(_LN_EPS))
                    stats.append((mean_v, rstd_v))
                for j in range(nv):
                    sl = pl.ds(j * _LANES, _LANES)
                    g = gamma_v[sl]
                    b = beta_v[sl]
                    for dt in range(_QUAD):
                        tk = t0 + dt
                        mean_v, rstd_v = stats[dt]
                        a = g * rstd_v
                        w[tk, sl] = (w[tk, sl] - mean_v) * a + b
                return carry

            lax.fori_loop(0, _CHUNK // _QUAD, quad_body, 0)
            pltpu.async_copy(w, out_hbm.at[pl.ds(base + off, _CHUNK)], sem_o)

        issue(0, wa, pa)

        def pair_body(cp, carry):
            c0 = cp * 2
            wait_gathers(wa)

            @pl.when(cp > 0)
            def _():
                drain_out(wb)

            issue(c0 + 1, wb, pb)
            compute(c0, wa, pa)
            wait_gathers(wb)

            @pl.when(cp + 1 < n_pairs)
            def _():
                drain_out(wa)
                issue(c0 + 2, wa, pa)

            compute(c0 + 1, wb, pb)
            return carry

        lax.fori_loop(0, n_pairs, pair_body, 0)
        drain_out(wa)
        drain_out(wb)

    return tie_kernel


def kernel(input_ids, token_type_ids, task_type_ids, word_emb, pos_emb,
           type_emb, task_emb, ln_gamma, ln_beta):
    b, s = input_ids.shape
    hidden = word_emb.shape[1]
    n_tok = b * s
    fn = _build(n_tok, s, hidden)
    out = fn(input_ids.reshape(-1).astype(jnp.int32),
             token_type_ids.reshape(-1).astype(jnp.int32),
             task_type_ids.reshape(-1).astype(jnp.int32),
             word_emb, pos_emb, type_emb, task_emb, ln_gamma, ln_beta)
    return out.reshape(b, s, hidden)
